# baseline (device time: 13303 ns/iter reference)
import jax
import jax.numpy as jnp
from jax import lax
from jax.experimental import pallas as pl
from jax.experimental.pallas import tpu as pltpu

N_DEV = 4


CHUNK_M = 512


def kernel(x):
    m_per, n = x.shape
    n_chunks = m_per // CHUNK_M

    def body(x_ref, out_ref, comm_ref, buf, copy_sems, send_sems, recv_sems):
        my = lax.axis_index("i")

        barrier_sem = pltpu.get_barrier_semaphore()
        for off in range(1, N_DEV):
            pl.semaphore_signal(
                barrier_sem,
                inc=1,
                device_id=((my + off) % N_DEV,),
                device_id_type=pl.DeviceIdType.MESH,
            )
        pl.semaphore_wait(barrier_sem, N_DEV - 1)

        def chunk_copy(i, slot):
            return pltpu.make_async_copy(
                x_ref.at[pl.ds(i * CHUNK_M, CHUNK_M)],
                buf.at[slot],
                copy_sems.at[slot],
            )

        chunk_copy(0, 0).start()
        acc = None
        for i in range(n_chunks):
            if i + 1 < n_chunks:
                chunk_copy(i + 1, (i + 1) % 2).start()
            chunk_copy(i, i % 2).wait()
            partial = jnp.sum(buf[i % 2], axis=0, keepdims=True)
            acc = partial if acc is None else acc + partial
        comm_ref[0, :, :] = acc

        rdmas = []
        for off in range(1, N_DEV):
            dst_slot = N_DEV - off
            rdma = pltpu.make_async_remote_copy(
                src_ref=comm_ref.at[0],
                dst_ref=comm_ref.at[dst_slot],
                send_sem=send_sems.at[off - 1],
                recv_sem=recv_sems.at[dst_slot],
                device_id=((my + off) % N_DEV,),
                device_id_type=pl.DeviceIdType.MESH,
            )
            rdma.start()
            rdmas.append(rdma)
        for rdma in rdmas:
            rdma.wait()

        out_ref[:, :] = (
            comm_ref[0, :, :]
            + comm_ref[1, :, :]
            + comm_ref[2, :, :]
            + comm_ref[3, :, :]
        )

    return pl.pallas_call(
        body,
        out_shape=jax.ShapeDtypeStruct((1, n), x.dtype),
        in_specs=[pl.BlockSpec(memory_space=pl.ANY)],
        out_specs=pl.BlockSpec(memory_space=pltpu.VMEM),
        scratch_shapes=[
            pltpu.VMEM((N_DEV, 1, n), x.dtype),
            pltpu.VMEM((2, CHUNK_M, n), x.dtype),
            pltpu.SemaphoreType.DMA((2,)),
            pltpu.SemaphoreType.DMA((N_DEV - 1,)),
            pltpu.SemaphoreType.DMA((N_DEV,)),
        ],
        compiler_params=pltpu.CompilerParams(collective_id=0),
    )(x)


# device time: 12146 ns/iter; 1.0953x vs baseline; 1.0953x over previous
import jax
import jax.numpy as jnp
from jax import lax
from jax.experimental import pallas as pl
from jax.experimental.pallas import tpu as pltpu

N_DEV = 4


def kernel(x):
    m_per, n = x.shape
    half = n // 2

    def body(x_ref, out_ref, comm_ref, send_sems, recv_sems):
        my = lax.axis_index("i")

        barrier_sem = pltpu.get_barrier_semaphore()
        for off in range(1, N_DEV):
            pl.semaphore_signal(
                barrier_sem,
                inc=1,
                device_id=((my + off) % N_DEV,),
                device_id_type=pl.DeviceIdType.MESH,
            )

        def send_half(h, lo):
            rdmas = []
            for off in range(1, N_DEV):
                dst_slot = N_DEV - off
                rdma = pltpu.make_async_remote_copy(
                    src_ref=comm_ref.at[0, :, pl.ds(lo, half)],
                    dst_ref=comm_ref.at[dst_slot, :, pl.ds(lo, half)],
                    send_sem=send_sems.at[off - 1, h],
                    recv_sem=recv_sems.at[dst_slot, h],
                    device_id=((my + off) % N_DEV,),
                    device_id_type=pl.DeviceIdType.MESH,
                )
                rdma.start()
                rdmas.append(rdma)
            return rdmas

        comm_ref[0, :, :half] = jnp.sum(
            x_ref[:, :half], axis=0, keepdims=True
        )
        pl.semaphore_wait(barrier_sem, N_DEV - 1)
        rdmas_a = send_half(0, 0)

        comm_ref[0, :, half:] = jnp.sum(
            x_ref[:, half:], axis=0, keepdims=True
        )
        rdmas_b = send_half(1, half)

        for rdma in rdmas_a + rdmas_b:
            rdma.wait()

        out_ref[:, :] = (
            comm_ref[0, :, :]
            + comm_ref[1, :, :]
            + comm_ref[2, :, :]
            + comm_ref[3, :, :]
        )

    return pl.pallas_call(
        body,
        out_shape=jax.ShapeDtypeStruct((1, n), x.dtype),
        in_specs=[pl.BlockSpec(memory_space=pltpu.VMEM)],
        out_specs=pl.BlockSpec(memory_space=pltpu.VMEM),
        scratch_shapes=[
            pltpu.VMEM((N_DEV, 1, n), x.dtype),
            pltpu.SemaphoreType.DMA((N_DEV - 1, 2)),
            pltpu.SemaphoreType.DMA((N_DEV, 2)),
        ],
        compiler_params=pltpu.CompilerParams(collective_id=0),
    )(x)


# device time: 11675 ns/iter; 1.1394x vs baseline; 1.0403x over previous
import jax
import jax.numpy as jnp
from jax import lax
from jax.experimental import pallas as pl
from jax.experimental.pallas import tpu as pltpu

N_DEV = 4


def kernel(x):
    m_per, n = x.shape

    def body(x_ref, out_ref, comm_ref, send_sems, recv_sems):
        my = lax.axis_index("i")

        barrier_sem = pltpu.get_barrier_semaphore()
        for off in range(1, N_DEV):
            pl.semaphore_signal(
                barrier_sem,
                inc=1,
                device_id=((my + off) % N_DEV,),
                device_id_type=pl.DeviceIdType.MESH,
            )

        comm_ref[0, :, :] = jnp.sum(x_ref[:, :], axis=0, keepdims=True)

        pl.semaphore_wait(barrier_sem, N_DEV - 1)

        rdmas = []
        for off in range(1, N_DEV):
            dst_slot = N_DEV - off
            rdma = pltpu.make_async_remote_copy(
                src_ref=comm_ref.at[0],
                dst_ref=comm_ref.at[dst_slot],
                send_sem=send_sems.at[off - 1],
                recv_sem=recv_sems.at[dst_slot],
                device_id=((my + off) % N_DEV,),
                device_id_type=pl.DeviceIdType.MESH,
            )
            rdma.start()
            rdmas.append(rdma)
        for rdma in rdmas:
            rdma.wait()

        out_ref[:, :] = (
            comm_ref[0, :, :]
            + comm_ref[1, :, :]
            + comm_ref[2, :, :]
            + comm_ref[3, :, :]
        )

    return pl.pallas_call(
        body,
        out_shape=jax.ShapeDtypeStruct((1, n), x.dtype),
        in_specs=[pl.BlockSpec(memory_space=pltpu.VMEM)],
        out_specs=pl.BlockSpec(memory_space=pltpu.VMEM),
        scratch_shapes=[
            pltpu.VMEM((N_DEV, 1, n), x.dtype),
            pltpu.SemaphoreType.DMA((N_DEV - 1,)),
            pltpu.SemaphoreType.DMA((N_DEV,)),
        ],
        compiler_params=pltpu.CompilerParams(collective_id=0),
    )(x)
